# Initial kernel scaffold; baseline (speedup 1.0000x reference)
import functools

import jax
import jax.numpy as jnp
from jax import lax
from jax.experimental import pallas as pl
from jax.experimental.pallas import tpu as pltpu
from jax.experimental.pallas import tpu_sc as plsc

N = 10000
E = 160000
D = 128
HID = 1024
C = 128
B = 8192
EPS = 1e-10

NC, NS = 2, 16
NW = NC * NS
EW = E // NW
K = 128
STEPS = (EW + K - 1) // K
EWP = STEPS * K
TAIL = EW - (STEPS - 1) * K
NPAD = N + 8
RPS = N // NS
HW = 16
RB = 500
GRID = N // RB
QB = 512
BF = jnp.bfloat16
F32 = jnp.float32


def _sc_histogram(dst3, vals_full_h, vals_tail_h, zrows_h):
    mesh = plsc.VectorSubcoreMesh(core_axis_name="c", subcore_axis_name="s")

    @functools.partial(
        pl.kernel,
        out_type=jax.ShapeDtypeStruct((NC, N, HW), F32),
        mesh=mesh,
        scratch_types=[
            pltpu.VMEM((STEPS, K), jnp.int32),
            pltpu.VMEM((K, HW), F32),
            pltpu.VMEM((K, HW), F32),
            pltpu.VMEM_SHARED((N, HW), F32),
        ],
    )
    def run(dst3_h, vf_h, vt_h, zr_h, out_h, dstv, vfull, vtail, acc):
        c = lax.axis_index("c")
        s = lax.axis_index("s")
        w = c * NS + s
        pltpu.sync_copy(dst3_h.at[w], dstv)
        pltpu.sync_copy(vf_h, vfull)
        pltpu.sync_copy(vt_h, vtail)
        r0 = s * RPS
        pltpu.sync_copy(zr_h, acc.at[pl.ds(r0, RPS)])
        plsc.subcore_barrier()

        @pl.loop(0, STEPS - 1)
        def _(j):
            pltpu.sync_copy(vfull, acc.at[dstv.at[j]], add=True)

        pltpu.sync_copy(vtail, acc.at[dstv.at[STEPS - 1]], add=True)
        plsc.subcore_barrier()
        pltpu.sync_copy(acc.at[pl.ds(r0, RPS)], out_h.at[c, pl.ds(r0, RPS)])

    return run(dst3, vals_full_h, vals_tail_h, zrows_h)


def _make_propagate(T):
    mesh = plsc.VectorSubcoreMesh(core_axis_name="c", subcore_axis_name="s")

    @functools.partial(
        pl.kernel,
        out_type=jax.ShapeDtypeStruct((NC, T, N, D), F32),
        mesh=mesh,
        scratch_types=[
            pltpu.VMEM((STEPS, K), jnp.int32),
            pltpu.VMEM((STEPS, K), jnp.int32),
            pltpu.VMEM((K, D), F32),
            pltpu.VMEM_SHARED((N, D), F32),
            pltpu.SemaphoreType.DMA,
        ],
    )
    def run(*refs):
        tabs = refs[:T]
        src3_h, dst3_h, zr_h, out_h = refs[T:T + 4]
        srcv, dstv, rows, acc, sem = refs[T + 4:]
        c = lax.axis_index("c")
        s = lax.axis_index("s")
        w = c * NS + s
        pltpu.sync_copy(src3_h.at[w], srcv)
        pltpu.sync_copy(dst3_h.at[w], dstv)
        r0 = s * RPS
        for t in range(T):
            pltpu.sync_copy(zr_h, acc.at[pl.ds(r0, RPS)])
            plsc.subcore_barrier()

            @pl.loop(0, STEPS)
            def _(j):
                pltpu.async_copy(tabs[t].at[srcv.at[j]], rows, sem).wait()
                pltpu.sync_copy(rows, acc.at[dstv.at[j]], add=True)

            plsc.subcore_barrier()
            pltpu.sync_copy(acc.at[pl.ds(r0, RPS)],
                            out_h.at[c, t, pl.ds(r0, RPS)])
            plsc.subcore_barrier()

    return run


_propagate1 = _make_propagate(1)
_propagate2 = _make_propagate(2)


def _sc_gather_pairs(tp, u3, v3):
    mesh = plsc.VectorSubcoreMesh(core_axis_name="c", subcore_axis_name="s")
    PW = B // NW // K

    @functools.partial(
        pl.kernel,
        out_type=(jax.ShapeDtypeStruct((B, 2 * C), F32),
                  jax.ShapeDtypeStruct((B, 2 * C), F32)),
        mesh=mesh,
        scratch_types=[
            pltpu.VMEM((PW, K), jnp.int32),
            pltpu.VMEM((K, 2 * C), F32),
            pltpu.SemaphoreType.DMA,
        ],
    )
    def run(tp_h, u3_h, v3_h, ru_h, rv_h, idxv, rows, sem):
        c = lax.axis_index("c")
        s = lax.axis_index("s")
        w = c * NS + s
        for idx_h, out_h in ((u3_h, ru_h), (v3_h, rv_h)):
            pltpu.sync_copy(idx_h.at[w], idxv)
            for j in range(PW):
                pltpu.async_copy(tp_h.at[idxv.at[j]], rows, sem).wait()
                pltpu.sync_copy(rows, out_h.at[pl.ds(w * PW * K + j * K, K)])

    return run(tp, u3, v3)


def _tc_scale(degp, x):
    def body(degp_ref, x_ref, xs_ref, dinv_ref):
        deg = degp_ref[0][:, 0:1] + degp_ref[1][:, 0:1] + 1.0
        dinv = lax.rsqrt(deg)
        xs_ref[...] = x_ref[...] * dinv
        dinv_ref[...] = jnp.broadcast_to(dinv, (RB, D))

    return pl.pallas_call(
        body,
        grid=(GRID,),
        in_specs=[
            pl.BlockSpec((NC, RB, HW), lambda i: (0, i, 0)),
            pl.BlockSpec((RB, D), lambda i: (i, 0)),
        ],
        out_specs=[
            pl.BlockSpec((RB, D), lambda i: (i, 0)),
            pl.BlockSpec((RB, D), lambda i: (i, 0)),
        ],
        out_shape=[jax.ShapeDtypeStruct((N, D), F32),
                   jax.ShapeDtypeStruct((N, D), F32)],
    )(degp, x)


def _tc_dense(xs, pp, dinv, w1t, b1t, w2t, w1p, b1p, w2p):
    def body(xs_ref, pp_ref, dinv_ref, w1t_ref, b1t_ref, w2t_ref,
             w1p_ref, b1p_ref, w2p_ref, gts_ref, gps_ref):
        dinv = dinv_ref[...]
        z = (dinv * (pp_ref[0, 0] + pp_ref[1, 0] + xs_ref[...])).astype(BF)
        ht = jnp.maximum(
            jnp.dot(z, w1t_ref[...], preferred_element_type=F32)
            + b1t_ref[...], 0.0).astype(BF)
        gts_ref[...] = jnp.dot(ht, w2t_ref[...],
                               preferred_element_type=F32) * dinv
        hp = jnp.maximum(
            jnp.dot(z, w1p_ref[...], preferred_element_type=F32)
            + b1p_ref[...], 0.0).astype(BF)
        gps_ref[...] = jnp.dot(hp, w2p_ref[...],
                               preferred_element_type=F32) * dinv

    return pl.pallas_call(
        body,
        grid=(GRID,),
        in_specs=[
            pl.BlockSpec((RB, D), lambda i: (i, 0)),
            pl.BlockSpec((NC, 1, RB, D), lambda i: (0, 0, i, 0)),
            pl.BlockSpec((RB, D), lambda i: (i, 0)),
            pl.BlockSpec((D, HID), lambda i: (0, 0)),
            pl.BlockSpec((1, HID), lambda i: (0, 0)),
            pl.BlockSpec((HID, C), lambda i: (0, 0)),
            pl.BlockSpec((D, HID), lambda i: (0, 0)),
            pl.BlockSpec((1, HID), lambda i: (0, 0)),
            pl.BlockSpec((HID, C), lambda i: (0, 0)),
        ],
        out_specs=[
            pl.BlockSpec((RB, C), lambda i: (i, 0)),
            pl.BlockSpec((RB, C), lambda i: (i, 0)),
        ],
        out_shape=[jax.ShapeDtypeStruct((N, C), F32),
                   jax.ShapeDtypeStruct((N, C), F32)],
    )(xs, pp, dinv, w1t, b1t, w2t, w1p, b1p, w2p)


def _tc_heads(qp, gts, gps, dinv, b2t, b2p):
    def body(qp_ref, gts_ref, gps_ref, dinv_ref, b2t_ref, b2p_ref, tp_ref):
        dinv = dinv_ref[...]
        st = dinv * (qp_ref[0, 0] + qp_ref[1, 0] + gts_ref[...]) + b2t_ref[...]
        st = st - jnp.max(st, axis=1, keepdims=True)
        et = jnp.exp(st)
        theta = et / jnp.sum(et, axis=1, keepdims=True)
        sp = dinv * (qp_ref[0, 1] + qp_ref[1, 1] + gps_ref[...]) + b2p_ref[...]
        phi = 1.0 / (1.0 + jnp.exp(-sp))
        tp_ref[...] = jnp.concatenate([theta, phi], axis=1)

    return pl.pallas_call(
        body,
        grid=(GRID,),
        in_specs=[
            pl.BlockSpec((NC, 2, RB, C), lambda i: (0, 0, i, 0)),
            pl.BlockSpec((RB, C), lambda i: (i, 0)),
            pl.BlockSpec((RB, C), lambda i: (i, 0)),
            pl.BlockSpec((RB, D), lambda i: (i, 0)),
            pl.BlockSpec((1, C), lambda i: (0, 0)),
            pl.BlockSpec((1, C), lambda i: (0, 0)),
        ],
        out_specs=pl.BlockSpec((RB, 2 * C), lambda i: (i, 0)),
        out_shape=jax.ShapeDtypeStruct((N, 2 * C), F32),
    )(qp, gts, gps, dinv, b2t, b2p)


def _tc_posterior(ru, rv, etap):
    def body(ru_ref, rv_ref, etap_ref, q_ref, p_ref, eta_ref):
        ex = jnp.exp(2.0 * etap_ref[...])
        eta = (ex - 1.0) / (ex + 1.0)
        ae = jnp.abs(eta)
        tu, pu = ru_ref[:, :C], ru_ref[:, C:]
        tv, pv = rv_ref[:, :C], rv_ref[:, C:]
        p = ae * tu * tv + (1.0 - ae) * pu * pv + EPS
        alpha = 1.0 + ae
        prior = alpha / jnp.sum(alpha)
        q = p * prior
        q_ref[...] = q / jnp.sum(q, axis=1, keepdims=True)
        p_ref[...] = p
        eta_ref[...] = eta

    return pl.pallas_call(
        body,
        grid=(B // QB,),
        in_specs=[
            pl.BlockSpec((QB, 2 * C), lambda i: (i, 0)),
            pl.BlockSpec((QB, 2 * C), lambda i: (i, 0)),
            pl.BlockSpec((1, C), lambda i: (0, 0)),
        ],
        out_specs=[
            pl.BlockSpec((QB, C), lambda i: (i, 0)),
            pl.BlockSpec((QB, C), lambda i: (i, 0)),
            pl.BlockSpec((1, C), lambda i: (0, 0)),
        ],
        out_shape=[jax.ShapeDtypeStruct((B, C), F32),
                   jax.ShapeDtypeStruct((B, C), F32),
                   jax.ShapeDtypeStruct((1, C), F32)],
    )(ru, rv, etap)


def kernel(x, edge_index, u, v, W1t, b1t, W2t, b2t, W1p, b1p, W2p, b2p,
           eta_param):
    src = edge_index[0]
    dst = edge_index[1]
    src3 = jnp.pad(src.reshape(NW, EW), ((0, 0), (0, EWP - EW)),
                   constant_values=N).reshape(NW, STEPS, K)
    dst3 = jnp.pad(dst.reshape(NW, EW), ((0, 0), (0, EWP - EW)),
                   constant_values=0).reshape(NW, STEPS, K)
    u3 = u.reshape(NW, B // NW // K, K)
    v3 = v.reshape(NW, B // NW // K, K)

    vals_full = jnp.ones((K, HW), F32)
    vals_tail = jnp.where(jnp.arange(K)[:, None] < TAIL, 1.0,
                          0.0) * jnp.ones((1, HW), F32)
    zrows_hw = jnp.zeros((RPS, HW), F32)
    zrows_d = jnp.zeros((RPS, D), F32)
    zpad = jnp.zeros((NPAD - N, D), F32)

    degp = _sc_histogram(dst3, vals_full, vals_tail, zrows_hw)
    xs, dinv = _tc_scale(degp, x)
    pp = _propagate1(jnp.concatenate([xs, zpad], axis=0), src3, dst3, zrows_d)
    gts, gps = _tc_dense(xs, pp, dinv,
                         W1t.astype(BF), b1t.reshape(1, HID), W2t.astype(BF),
                         W1p.astype(BF), b1p.reshape(1, HID), W2p.astype(BF))
    qp = _propagate2(jnp.concatenate([gts, zpad], axis=0),
                     jnp.concatenate([gps, zpad], axis=0), src3, dst3, zrows_d)
    tp = _tc_heads(qp, gts, gps, dinv, b2t.reshape(1, C), b2p.reshape(1, C))
    ru, rv = _sc_gather_pairs(tp, u3, v3)
    q_probs, p_probs, eta = _tc_posterior(ru, rv, eta_param.reshape(1, C))
    return (q_probs, p_probs, eta.reshape(C))


# speedup vs baseline: 12.3406x; 12.3406x over previous
import functools

import jax
import jax.numpy as jnp
from jax import lax
from jax.experimental import pallas as pl
from jax.experimental.pallas import tpu as pltpu
from jax.experimental.pallas import tpu_sc as plsc

N = 10000
E = 160000
D = 128
HID = 1024
C = 128
B = 8192
EPS = 1e-10

NC, NS = 2, 16
NW = NC * NS
EW = E // NW
K = 128
STEPS = (EW + K - 1) // K
EWP = STEPS * K
TAIL = EW - (STEPS - 1) * K
NPAD = N + 8
NACC = 10240
RPS = NACC // NS
HW = 128
RB = 1000
GRID = N // RB
QB = 512
BF = jnp.bfloat16
F32 = jnp.float32


def _sc_histogram(dst3, vals_full_h, vals_tail_h, zrows_h):
    mesh = plsc.VectorSubcoreMesh(core_axis_name="c", subcore_axis_name="s")

    @functools.partial(
        pl.kernel,
        out_type=jax.ShapeDtypeStruct((NC, NACC, HW), F32),
        mesh=mesh,
        scratch_types=[
            pltpu.VMEM((STEPS, K), jnp.int32),
            pltpu.VMEM((K, HW), F32),
            pltpu.VMEM((K, HW), F32),
            pltpu.VMEM_SHARED((NACC, HW), F32),
        ],
    )
    def run(dst3_h, vf_h, vt_h, zr_h, out_h, dstv, vfull, vtail, acc):
        c = lax.axis_index("c")
        s = lax.axis_index("s")
        w = c * NS + s
        pltpu.sync_copy(dst3_h.at[w], dstv)
        pltpu.sync_copy(vf_h, vfull)
        pltpu.sync_copy(vt_h, vtail)
        r0 = s * RPS
        pltpu.sync_copy(zr_h, acc.at[pl.ds(r0, RPS)])
        plsc.subcore_barrier()

        @pl.loop(0, STEPS - 1)
        def _(j):
            pltpu.sync_copy(vfull, acc.at[dstv.at[j]], add=True)

        pltpu.sync_copy(vtail, acc.at[dstv.at[STEPS - 1]], add=True)
        plsc.subcore_barrier()
        pltpu.sync_copy(acc.at[pl.ds(r0, RPS)], out_h.at[c, pl.ds(r0, RPS)])

    return run(dst3, vals_full_h, vals_tail_h, zrows_h)


def _make_propagate(T):
    mesh = plsc.VectorSubcoreMesh(core_axis_name="c", subcore_axis_name="s")

    @functools.partial(
        pl.kernel,
        out_type=jax.ShapeDtypeStruct((NC, T, NACC, D), F32),
        mesh=mesh,
        scratch_types=[
            pltpu.VMEM((STEPS, K), jnp.int32),
            pltpu.VMEM((STEPS, K), jnp.int32),
            pltpu.VMEM((K, D), F32),
            pltpu.VMEM_SHARED((NACC, D), F32),
            pltpu.SemaphoreType.DMA,
        ],
    )
    def run(*refs):
        tabs = refs[:T]
        src3_h, dst3_h, zr_h, out_h = refs[T:T + 4]
        srcv, dstv, rows, acc, sem = refs[T + 4:]
        c = lax.axis_index("c")
        s = lax.axis_index("s")
        w = c * NS + s
        pltpu.sync_copy(src3_h.at[w], srcv)
        pltpu.sync_copy(dst3_h.at[w], dstv)
        r0 = s * RPS
        for t in range(T):
            pltpu.sync_copy(zr_h, acc.at[pl.ds(r0, RPS)])
            plsc.subcore_barrier()

            @pl.loop(0, STEPS)
            def _(j):
                pltpu.async_copy(tabs[t].at[srcv.at[j]], rows, sem).wait()
                pltpu.sync_copy(rows, acc.at[dstv.at[j]], add=True)

            plsc.subcore_barrier()
            pltpu.sync_copy(acc.at[pl.ds(r0, RPS)],
                            out_h.at[c, t, pl.ds(r0, RPS)])
            plsc.subcore_barrier()

    return run


_propagate1 = _make_propagate(1)
_propagate2 = _make_propagate(2)


def _sc_gather_pairs(tp, u3, v3):
    mesh = plsc.VectorSubcoreMesh(core_axis_name="c", subcore_axis_name="s")
    PW = B // NW // K

    @functools.partial(
        pl.kernel,
        out_type=(jax.ShapeDtypeStruct((B, 2 * C), F32),
                  jax.ShapeDtypeStruct((B, 2 * C), F32)),
        mesh=mesh,
        scratch_types=[
            pltpu.VMEM((PW, K), jnp.int32),
            pltpu.VMEM((K, 2 * C), F32),
            pltpu.SemaphoreType.DMA,
        ],
    )
    def run(tp_h, u3_h, v3_h, ru_h, rv_h, idxv, rows, sem):
        c = lax.axis_index("c")
        s = lax.axis_index("s")
        w = c * NS + s
        for idx_h, out_h in ((u3_h, ru_h), (v3_h, rv_h)):
            pltpu.sync_copy(idx_h.at[w], idxv)
            for j in range(PW):
                pltpu.async_copy(tp_h.at[idxv.at[j]], rows, sem).wait()
                pltpu.sync_copy(rows, out_h.at[pl.ds(w * PW * K + j * K, K)])

    return run(tp, u3, v3)


def _tc_scale(degp, x):
    def body(degp_ref, x_ref, xs_ref, dinv_ref):
        deg = degp_ref[0][:, 0:1] + degp_ref[1][:, 0:1] + 1.0
        dinv = lax.rsqrt(deg)
        xs_ref[...] = x_ref[...] * dinv
        dinv_ref[...] = jnp.broadcast_to(dinv, (RB, D))

    return pl.pallas_call(
        body,
        grid=(GRID,),
        in_specs=[
            pl.BlockSpec((NC, RB, HW), lambda i: (0, i, 0)),
            pl.BlockSpec((RB, D), lambda i: (i, 0)),
        ],
        out_specs=[
            pl.BlockSpec((RB, D), lambda i: (i, 0)),
            pl.BlockSpec((RB, D), lambda i: (i, 0)),
        ],
        out_shape=[jax.ShapeDtypeStruct((N, D), F32),
                   jax.ShapeDtypeStruct((N, D), F32)],
    )(degp, x)


def _tc_dense(xs, pp, dinv, w1t, b1t, w2t, w1p, b1p, w2p):
    def body(xs_ref, pp_ref, dinv_ref, w1t_ref, b1t_ref, w2t_ref,
             w1p_ref, b1p_ref, w2p_ref, gts_ref, gps_ref):
        dinv = dinv_ref[...]
        z = (dinv * (pp_ref[0, 0] + pp_ref[1, 0] + xs_ref[...])).astype(BF)
        ht = jnp.maximum(
            jnp.dot(z, w1t_ref[...], preferred_element_type=F32)
            + b1t_ref[...], 0.0).astype(BF)
        gts_ref[...] = jnp.dot(ht, w2t_ref[...],
                               preferred_element_type=F32) * dinv
        hp = jnp.maximum(
            jnp.dot(z, w1p_ref[...], preferred_element_type=F32)
            + b1p_ref[...], 0.0).astype(BF)
        gps_ref[...] = jnp.dot(hp, w2p_ref[...],
                               preferred_element_type=F32) * dinv

    return pl.pallas_call(
        body,
        grid=(GRID,),
        in_specs=[
            pl.BlockSpec((RB, D), lambda i: (i, 0)),
            pl.BlockSpec((NC, 1, RB, D), lambda i: (0, 0, i, 0)),
            pl.BlockSpec((RB, D), lambda i: (i, 0)),
            pl.BlockSpec((D, HID), lambda i: (0, 0)),
            pl.BlockSpec((1, HID), lambda i: (0, 0)),
            pl.BlockSpec((HID, C), lambda i: (0, 0)),
            pl.BlockSpec((D, HID), lambda i: (0, 0)),
            pl.BlockSpec((1, HID), lambda i: (0, 0)),
            pl.BlockSpec((HID, C), lambda i: (0, 0)),
        ],
        out_specs=[
            pl.BlockSpec((RB, C), lambda i: (i, 0)),
            pl.BlockSpec((RB, C), lambda i: (i, 0)),
        ],
        out_shape=[jax.ShapeDtypeStruct((N, C), F32),
                   jax.ShapeDtypeStruct((N, C), F32)],
    )(xs, pp, dinv, w1t, b1t, w2t, w1p, b1p, w2p)


def _tc_heads(qp, gts, gps, dinv, b2t, b2p):
    def body(qp_ref, gts_ref, gps_ref, dinv_ref, b2t_ref, b2p_ref, tp_ref):
        dinv = dinv_ref[...]
        st = dinv * (qp_ref[0, 0] + qp_ref[1, 0] + gts_ref[...]) + b2t_ref[...]
        st = st - jnp.max(st, axis=1, keepdims=True)
        et = jnp.exp(st)
        theta = et / jnp.sum(et, axis=1, keepdims=True)
        sp = dinv * (qp_ref[0, 1] + qp_ref[1, 1] + gps_ref[...]) + b2p_ref[...]
        phi = 1.0 / (1.0 + jnp.exp(-sp))
        tp_ref[...] = jnp.concatenate([theta, phi], axis=1)

    return pl.pallas_call(
        body,
        grid=(GRID,),
        in_specs=[
            pl.BlockSpec((NC, 2, RB, C), lambda i: (0, 0, i, 0)),
            pl.BlockSpec((RB, C), lambda i: (i, 0)),
            pl.BlockSpec((RB, C), lambda i: (i, 0)),
            pl.BlockSpec((RB, D), lambda i: (i, 0)),
            pl.BlockSpec((1, C), lambda i: (0, 0)),
            pl.BlockSpec((1, C), lambda i: (0, 0)),
        ],
        out_specs=pl.BlockSpec((RB, 2 * C), lambda i: (i, 0)),
        out_shape=jax.ShapeDtypeStruct((N, 2 * C), F32),
    )(qp, gts, gps, dinv, b2t, b2p)


def _tc_posterior(ru, rv, etap):
    def body(ru_ref, rv_ref, etap_ref, q_ref, p_ref, eta_ref):
        ex = jnp.exp(2.0 * etap_ref[...])
        eta = (ex - 1.0) / (ex + 1.0)
        ae = jnp.abs(eta)
        tu, pu = ru_ref[:, :C], ru_ref[:, C:]
        tv, pv = rv_ref[:, :C], rv_ref[:, C:]
        p = ae * tu * tv + (1.0 - ae) * pu * pv + EPS
        alpha = 1.0 + ae
        prior = alpha / jnp.sum(alpha)
        q = p * prior
        q_ref[...] = q / jnp.sum(q, axis=1, keepdims=True)
        p_ref[...] = p
        eta_ref[...] = eta

    return pl.pallas_call(
        body,
        grid=(B // QB,),
        in_specs=[
            pl.BlockSpec((QB, 2 * C), lambda i: (i, 0)),
            pl.BlockSpec((QB, 2 * C), lambda i: (i, 0)),
            pl.BlockSpec((1, C), lambda i: (0, 0)),
        ],
        out_specs=[
            pl.BlockSpec((QB, C), lambda i: (i, 0)),
            pl.BlockSpec((QB, C), lambda i: (i, 0)),
            pl.BlockSpec((1, C), lambda i: (0, 0)),
        ],
        out_shape=[jax.ShapeDtypeStruct((B, C), F32),
                   jax.ShapeDtypeStruct((B, C), F32),
                   jax.ShapeDtypeStruct((1, C), F32)],
    )(ru, rv, etap)


def kernel(x, edge_index, u, v, W1t, b1t, W2t, b2t, W1p, b1p, W2p, b2p,
           eta_param):
    src = edge_index[0]
    dst = edge_index[1]
    src3 = jnp.pad(src.reshape(NW, EW), ((0, 0), (0, EWP - EW)),
                   constant_values=N).reshape(NW, STEPS, K)
    dst3 = jnp.pad(dst.reshape(NW, EW), ((0, 0), (0, EWP - EW)),
                   constant_values=0).reshape(NW, STEPS, K)
    u3 = u.reshape(NW, B // NW // K, K)
    v3 = v.reshape(NW, B // NW // K, K)

    vals_full = jnp.ones((K, HW), F32)
    vals_tail = jnp.where(jnp.arange(K)[:, None] < TAIL, 1.0,
                          0.0) * jnp.ones((1, HW), F32)
    zrows_hw = jnp.zeros((RPS, HW), F32)
    zrows_d = jnp.zeros((RPS, D), F32)
    zpad = jnp.zeros((NPAD - N, D), F32)

    degp = _sc_histogram(dst3, vals_full, vals_tail, zrows_hw)
    xs, dinv = _tc_scale(degp, x)
    pp = _propagate1(jnp.concatenate([xs, zpad], axis=0), src3, dst3, zrows_d)
    gts, gps = _tc_dense(xs, pp, dinv,
                         W1t.astype(BF), b1t.reshape(1, HID), W2t.astype(BF),
                         W1p.astype(BF), b1p.reshape(1, HID), W2p.astype(BF))
    qp = _propagate2(jnp.concatenate([gts, zpad], axis=0),
                     jnp.concatenate([gps, zpad], axis=0), src3, dst3, zrows_d)
    tp = _tc_heads(qp, gts, gps, dinv, b2t.reshape(1, C), b2p.reshape(1, C))
    ru, rv = _sc_gather_pairs(tp, u3, v3)
    q_probs, p_probs, eta = _tc_posterior(ru, rv, eta_param.reshape(1, C))
    return (q_probs, p_probs, eta.reshape(C))


# speedup vs baseline: 13.5375x; 1.0970x over previous
import functools

import jax
import jax.numpy as jnp
from jax import lax
from jax.experimental import pallas as pl
from jax.experimental.pallas import tpu as pltpu
from jax.experimental.pallas import tpu_sc as plsc

N = 10000
E = 160000
D = 128
HID = 1024
C = 128
B = 8192
EPS = 1e-10

NC, NS = 2, 16
NW = NC * NS
EW = E // NW
K = 128
STEPS = (EW + K - 1) // K
EWP = STEPS * K
TAIL = EW - (STEPS - 1) * K
NACC = 10240
DUMMY = 10016
RPS = NACC // NS
HW = 128
RB = 1000
GRID = N // RB
QB = 512
BF = jnp.bfloat16
F32 = jnp.float32


def _sc_histogram(dst3, vals_full_h, zrows_h):
    mesh = plsc.VectorSubcoreMesh(core_axis_name="c", subcore_axis_name="s")

    @functools.partial(
        pl.kernel,
        out_type=jax.ShapeDtypeStruct((NC, NACC, HW), F32),
        mesh=mesh,
        scratch_types=[
            pltpu.VMEM((STEPS, K), jnp.int32),
            pltpu.VMEM((K, HW), F32),
            pltpu.VMEM_SHARED((NACC, HW), F32),
            pltpu.SemaphoreType.DMA,
        ],
    )
    def run(dst3_h, vf_h, zr_h, out_h, dstv, vfull, acc, sem):
        c = lax.axis_index("c")
        s = lax.axis_index("s")
        w = c * NS + s
        pltpu.sync_copy(dst3_h.at[w], dstv)
        pltpu.sync_copy(vf_h, vfull)
        r0 = s * RPS
        pltpu.sync_copy(zr_h, acc.at[pl.ds(r0, RPS)])
        plsc.subcore_barrier()

        @pl.loop(0, STEPS)
        def _(j):
            pltpu.async_copy(vfull, acc.at[dstv.at[j]], sem, add=True)

        @pl.loop(0, STEPS)
        def _(j):
            pltpu.make_async_copy(vfull, acc.at[dstv.at[0]], sem).wait()

        plsc.subcore_barrier()
        pltpu.sync_copy(acc.at[pl.ds(r0, RPS)], out_h.at[c, pl.ds(r0, RPS)])

    return run(dst3, vals_full_h, zrows_h)


def _make_propagate(T):
    mesh = plsc.VectorSubcoreMesh(core_axis_name="c", subcore_axis_name="s")

    NB = 2

    @functools.partial(
        pl.kernel,
        out_type=jax.ShapeDtypeStruct((NC, T, NACC, D), F32),
        mesh=mesh,
        scratch_types=[
            pltpu.VMEM((STEPS, K), jnp.int32),
            pltpu.VMEM((STEPS, K), jnp.int32),
            [pltpu.VMEM((K, D), F32)] * NB,
            pltpu.VMEM_SHARED((NACC, D), F32),
            [pltpu.SemaphoreType.DMA] * NB,
            [pltpu.SemaphoreType.DMA] * NB,
        ],
    )
    def run(*refs):
        tabs = refs[:T]
        src3_h, dst3_h, zr_h, out_h = refs[T:T + 4]
        srcv, dstv, rows, acc, gsem, ssem = refs[T + 4:]
        c = lax.axis_index("c")
        s = lax.axis_index("s")
        w = c * NS + s
        pltpu.sync_copy(src3_h.at[w], srcv)
        pltpu.sync_copy(dst3_h.at[w], dstv)
        r0 = s * RPS
        for t in range(T):
            tab = tabs[t]

            def fire_g(j, r):
                pltpu.async_copy(tab.at[srcv.at[j]], rows[r], gsem[r])

            def wait_g(r):
                pltpu.make_async_copy(tab.at[srcv.at[0]], rows[r],
                                      gsem[r]).wait()

            def fire_s(j, r):
                pltpu.async_copy(rows[r], acc.at[dstv.at[j]], ssem[r],
                                 add=True)

            def wait_s(r):
                pltpu.make_async_copy(rows[r], acc.at[dstv.at[0]],
                                      ssem[r]).wait()

            def step(j, r, refill):
                wait_g(r)
                fire_s(j, r)
                if refill:
                    r2 = (r + 1) % NB
                    wait_s(r2)
                    fire_g(j + 1, r2)

            pltpu.sync_copy(zr_h, acc.at[pl.ds(r0, RPS)])
            plsc.subcore_barrier()
            fire_g(0, 0)
            fire_g(1, 1)
            step(0, 0, False)
            step(1, 1, True)

            @pl.loop(1, STEPS // NB - 1)
            def _(i):
                j0 = i * NB
                step(j0, 0, True)
                step(j0 + 1, 1, True)

            step(STEPS - 2, 0, True)
            step(STEPS - 1, 1, False)
            for r in range(NB):
                wait_s(r)
            plsc.subcore_barrier()
            pltpu.sync_copy(acc.at[pl.ds(r0, RPS)],
                            out_h.at[c, t, pl.ds(r0, RPS)])
            plsc.subcore_barrier()

    return run


_propagate1 = _make_propagate(1)
_propagate2 = _make_propagate(2)


def _sc_gather_pairs(tp, u3, v3):
    mesh = plsc.VectorSubcoreMesh(core_axis_name="c", subcore_axis_name="s")
    PW = B // NW // K

    @functools.partial(
        pl.kernel,
        out_type=(jax.ShapeDtypeStruct((B, 2 * C), F32),
                  jax.ShapeDtypeStruct((B, 2 * C), F32)),
        mesh=mesh,
        scratch_types=[
            pltpu.VMEM((PW, K), jnp.int32),
            pltpu.VMEM((K, 2 * C), F32),
            pltpu.SemaphoreType.DMA,
        ],
    )
    def run(tp_h, u3_h, v3_h, ru_h, rv_h, idxv, rows, sem):
        c = lax.axis_index("c")
        s = lax.axis_index("s")
        w = c * NS + s
        for idx_h, out_h in ((u3_h, ru_h), (v3_h, rv_h)):
            pltpu.sync_copy(idx_h.at[w], idxv)
            for j in range(PW):
                pltpu.async_copy(tp_h.at[idxv.at[j]], rows, sem).wait()
                pltpu.sync_copy(rows, out_h.at[pl.ds(w * PW * K + j * K, K)])

    return run(tp, u3, v3)


def _tc_scale(degp, x):
    def body(degp_ref, x_ref, xs_ref, dinv_ref):
        deg = degp_ref[0][:, 0:1] + degp_ref[1][:, 0:1] + 1.0
        dinv = lax.rsqrt(deg)
        xs_ref[...] = x_ref[...] * dinv
        dinv_ref[...] = jnp.broadcast_to(dinv, (RB, D))

    return pl.pallas_call(
        body,
        grid=(GRID,),
        in_specs=[
            pl.BlockSpec((NC, RB, HW), lambda i: (0, i, 0)),
            pl.BlockSpec((RB, D), lambda i: (i, 0)),
        ],
        out_specs=[
            pl.BlockSpec((RB, D), lambda i: (i, 0)),
            pl.BlockSpec((RB, D), lambda i: (i, 0)),
        ],
        out_shape=[jax.ShapeDtypeStruct((N, D), F32),
                   jax.ShapeDtypeStruct((N, D), F32)],
    )(degp, x)


def _tc_dense(xs, pp, dinv, w1t, b1t, w2t, w1p, b1p, w2p):
    def body(xs_ref, pp_ref, dinv_ref, w1t_ref, b1t_ref, w2t_ref,
             w1p_ref, b1p_ref, w2p_ref, gts_ref, gps_ref):
        dinv = dinv_ref[...]
        z = (dinv * (pp_ref[0, 0] + pp_ref[1, 0] + xs_ref[...])).astype(BF)
        ht = jnp.maximum(
            jnp.dot(z, w1t_ref[...], preferred_element_type=F32)
            + b1t_ref[...], 0.0).astype(BF)
        gts_ref[...] = jnp.dot(ht, w2t_ref[...],
                               preferred_element_type=F32) * dinv
        hp = jnp.maximum(
            jnp.dot(z, w1p_ref[...], preferred_element_type=F32)
            + b1p_ref[...], 0.0).astype(BF)
        gps_ref[...] = jnp.dot(hp, w2p_ref[...],
                               preferred_element_type=F32) * dinv

    return pl.pallas_call(
        body,
        grid=(GRID,),
        in_specs=[
            pl.BlockSpec((RB, D), lambda i: (i, 0)),
            pl.BlockSpec((NC, 1, RB, D), lambda i: (0, 0, i, 0)),
            pl.BlockSpec((RB, D), lambda i: (i, 0)),
            pl.BlockSpec((D, HID), lambda i: (0, 0)),
            pl.BlockSpec((1, HID), lambda i: (0, 0)),
            pl.BlockSpec((HID, C), lambda i: (0, 0)),
            pl.BlockSpec((D, HID), lambda i: (0, 0)),
            pl.BlockSpec((1, HID), lambda i: (0, 0)),
            pl.BlockSpec((HID, C), lambda i: (0, 0)),
        ],
        out_specs=[
            pl.BlockSpec((RB, C), lambda i: (i, 0)),
            pl.BlockSpec((RB, C), lambda i: (i, 0)),
        ],
        out_shape=[jax.ShapeDtypeStruct((N, C), F32),
                   jax.ShapeDtypeStruct((N, C), F32)],
    )(xs, pp, dinv, w1t, b1t, w2t, w1p, b1p, w2p)


def _tc_heads(qp, gts, gps, dinv, b2t, b2p):
    def body(qp_ref, gts_ref, gps_ref, dinv_ref, b2t_ref, b2p_ref, tp_ref):
        dinv = dinv_ref[...]
        st = dinv * (qp_ref[0, 0] + qp_ref[1, 0] + gts_ref[...]) + b2t_ref[...]
        st = st - jnp.max(st, axis=1, keepdims=True)
        et = jnp.exp(st)
        theta = et / jnp.sum(et, axis=1, keepdims=True)
        sp = dinv * (qp_ref[0, 1] + qp_ref[1, 1] + gps_ref[...]) + b2p_ref[...]
        phi = 1.0 / (1.0 + jnp.exp(-sp))
        tp_ref[...] = jnp.concatenate([theta, phi], axis=1)

    return pl.pallas_call(
        body,
        grid=(GRID,),
        in_specs=[
            pl.BlockSpec((NC, 2, RB, C), lambda i: (0, 0, i, 0)),
            pl.BlockSpec((RB, C), lambda i: (i, 0)),
            pl.BlockSpec((RB, C), lambda i: (i, 0)),
            pl.BlockSpec((RB, D), lambda i: (i, 0)),
            pl.BlockSpec((1, C), lambda i: (0, 0)),
            pl.BlockSpec((1, C), lambda i: (0, 0)),
        ],
        out_specs=pl.BlockSpec((RB, 2 * C), lambda i: (i, 0)),
        out_shape=jax.ShapeDtypeStruct((N, 2 * C), F32),
    )(qp, gts, gps, dinv, b2t, b2p)


def _tc_posterior(ru, rv, etap):
    def body(ru_ref, rv_ref, etap_ref, q_ref, p_ref, eta_ref):
        ex = jnp.exp(2.0 * etap_ref[...])
        eta = (ex - 1.0) / (ex + 1.0)
        ae = jnp.abs(eta)
        tu, pu = ru_ref[:, :C], ru_ref[:, C:]
        tv, pv = rv_ref[:, :C], rv_ref[:, C:]
        p = ae * tu * tv + (1.0 - ae) * pu * pv + EPS
        alpha = 1.0 + ae
        prior = alpha / jnp.sum(alpha)
        q = p * prior
        q_ref[...] = q / jnp.sum(q, axis=1, keepdims=True)
        p_ref[...] = p
        eta_ref[...] = eta

    return pl.pallas_call(
        body,
        grid=(B // QB,),
        in_specs=[
            pl.BlockSpec((QB, 2 * C), lambda i: (i, 0)),
            pl.BlockSpec((QB, 2 * C), lambda i: (i, 0)),
            pl.BlockSpec((1, C), lambda i: (0, 0)),
        ],
        out_specs=[
            pl.BlockSpec((QB, C), lambda i: (i, 0)),
            pl.BlockSpec((QB, C), lambda i: (i, 0)),
            pl.BlockSpec((1, C), lambda i: (0, 0)),
        ],
        out_shape=[jax.ShapeDtypeStruct((B, C), F32),
                   jax.ShapeDtypeStruct((B, C), F32),
                   jax.ShapeDtypeStruct((1, C), F32)],
    )(ru, rv, etap)


def kernel(x, edge_index, u, v, W1t, b1t, W2t, b2t, W1p, b1p, W2p, b2p,
           eta_param):
    src = edge_index[0]
    dst = edge_index[1]
    src3 = jnp.pad(src.reshape(NW, EW), ((0, 0), (0, EWP - EW)),
                   constant_values=0).reshape(NW, STEPS, K)
    dst3 = jnp.pad(dst.reshape(NW, EW), ((0, 0), (0, EWP - EW)),
                   constant_values=DUMMY).reshape(NW, STEPS, K)
    u3 = u.reshape(NW, B // NW // K, K)
    v3 = v.reshape(NW, B // NW // K, K)

    vals_full = jnp.ones((K, HW), F32)
    zrows_hw = jnp.zeros((RPS, HW), F32)
    zrows_d = jnp.zeros((RPS, D), F32)

    degp = _sc_histogram(dst3, vals_full, zrows_hw)
    xs, dinv = _tc_scale(degp, x)
    pp = _propagate1(xs, src3, dst3, zrows_d)
    gts, gps = _tc_dense(xs, pp, dinv,
                         W1t.astype(BF), b1t.reshape(1, HID), W2t.astype(BF),
                         W1p.astype(BF), b1p.reshape(1, HID), W2p.astype(BF))
    qp = _propagate2(gts, gps, src3, dst3, zrows_d)
    tp = _tc_heads(qp, gts, gps, dinv, b2t.reshape(1, C), b2p.reshape(1, C))
    ru, rv = _sc_gather_pairs(tp, u3, v3)
    q_probs, p_probs, eta = _tc_posterior(ru, rv, eta_param.reshape(1, C))
    return (q_probs, p_probs, eta.reshape(C))
